# Initial kernel scaffold; baseline (speedup 1.0000x reference)
import math

import jax
import jax.numpy as jnp
from jax import lax
from jax.experimental import pallas as pl
from jax.experimental.pallas import tpu as pltpu
from jax.experimental.pallas import tpu_sc as plsc

D = 32
SCALE = math.sqrt(32.0)

_info = plsc.get_sparse_core_info()
NC, NS, L = _info.num_cores, _info.num_subcores, _info.num_lanes
NW = NC * NS

B_TOTAL = 4096 * 200
B_PER_W = B_TOTAL // NW
CHUNK = 2560
NCHUNK = B_PER_W // CHUNK
ROWS_U = 8


def _body(table_hbm, idx_hbm, out_hbm, idx_v, rows_v, sem):
    wid = lax.axis_index("s") * NC + lax.axis_index("c")
    base = wid * B_PER_W

    def chunk_step(c, carry):
        off = base + c * CHUNK
        pltpu.sync_copy(idx_hbm.at[pl.ds(off, CHUNK)], idx_v)
        pltpu.async_copy(table_hbm.at[idx_v], rows_v, sem).wait()

        def scale_step(i, carry2):
            r0 = i * ROWS_U
            for u in range(ROWS_U):
                for h in range(D // L):
                    v = rows_v[r0 + u, pl.ds(h * L, L)]
                    rows_v[r0 + u, pl.ds(h * L, L)] = v * SCALE
            return carry2

        lax.fori_loop(0, CHUNK // ROWS_U, scale_step, 0)
        pltpu.sync_copy(rows_v, out_hbm.at[pl.ds(off, CHUNK)])
        return carry

    lax.fori_loop(0, NCHUNK, chunk_step, 0)


def kernel(x, table):
    xf = x.reshape(-1).astype(jnp.int32)
    mesh = plsc.VectorSubcoreMesh(core_axis_name="c", subcore_axis_name="s")
    out = pl.kernel(
        _body,
        mesh=mesh,
        out_type=jax.ShapeDtypeStruct((B_TOTAL, D), jnp.float32),
        scratch_types=[
            pltpu.VMEM((CHUNK,), jnp.int32),
            pltpu.VMEM((CHUNK, D), jnp.float32),
            pltpu.SemaphoreType.DMA,
        ],
    )(table, xf)
    return out.reshape(x.shape[0], x.shape[1], D)


# speedup vs baseline: 1.4287x; 1.4287x over previous
import math

import jax
import jax.numpy as jnp
from jax import lax
from jax.experimental import pallas as pl
from jax.experimental.pallas import tpu as pltpu
from jax.experimental.pallas import tpu_sc as plsc

D = 32
SCALE = math.sqrt(32.0)

_info = plsc.get_sparse_core_info()
NC, NS, L = _info.num_cores, _info.num_subcores, _info.num_lanes
NW = NC * NS

B_TOTAL = 4096 * 200
B_PER_W = B_TOTAL // NW
CHUNK = 2560
NCHUNK = B_PER_W // CHUNK
ROWS_U = 8


def _body(table_hbm, idx_hbm, out_hbm, idx_v, rows_v, sem):
    wid = lax.axis_index("s") * NC + lax.axis_index("c")
    base = wid * B_PER_W

    def chunk_step(c, carry):
        off = base + c * CHUNK
        pltpu.sync_copy(idx_hbm.at[pl.ds(off, CHUNK)], idx_v)
        pltpu.async_copy(table_hbm.at[idx_v], rows_v, sem).wait()

        def scale_step(i, carry2):
            r0 = i * ROWS_U
            for u in range(ROWS_U):
                for h in range(D // L):
                    v = rows_v[r0 + u, pl.ds(h * L, L)]
                    rows_v[r0 + u, pl.ds(h * L, L)] = v * SCALE
            return carry2

        lax.fori_loop(0, CHUNK // ROWS_U, scale_step, 0)
        pltpu.sync_copy(rows_v, out_hbm.at[pl.ds(off, CHUNK)])
        return carry

    lax.fori_loop(0, NCHUNK, chunk_step, 0)


def kernel(x, table):
    xf = x.reshape(-1).astype(jnp.int32)
    mesh = plsc.VectorSubcoreMesh(core_axis_name="c", subcore_axis_name="s")
    out = pl.kernel(
        _body,
        mesh=mesh,
        out_type=jax.ShapeDtypeStruct((B_TOTAL, D), jnp.float32),
        scratch_types=[
            pltpu.VMEM((CHUNK,), jnp.int32),
            pltpu.VMEM((CHUNK, D), jnp.float32),
            pltpu.SemaphoreType.DMA,
        ],
        compiler_params=pltpu.CompilerParams(use_tc_tiling_on_sc=False),
    )(table, xf)
    return out.reshape(x.shape[0], x.shape[1], D)


# speedup vs baseline: 1.4769x; 1.0337x over previous
import math

import jax
import jax.numpy as jnp
from jax import lax
from jax.experimental import pallas as pl
from jax.experimental.pallas import tpu as pltpu
from jax.experimental.pallas import tpu_sc as plsc

D = 32
SCALE = math.sqrt(32.0)

_info = plsc.get_sparse_core_info()
NC, NS, L = _info.num_cores, _info.num_subcores, _info.num_lanes
NW = NC * NS

B_TOTAL = 4096 * 200
B_PER_W = B_TOTAL // NW
NBUF = 3
CHUNK = 800
NCHUNK = B_PER_W // CHUNK
ROWS_U = 8


def _body(table_hbm, idx_hbm, out_hbm, idx_all, rows, gsems, osems):
    wid = lax.axis_index("s") * NC + lax.axis_index("c")
    base = wid * B_PER_W

    pltpu.sync_copy(idx_hbm.at[pl.ds(base, B_PER_W)], idx_all)

    def gather_copy(c):
        b = c % NBUF
        idx_slice = idx_all.at[pl.ds(c * CHUNK, CHUNK)]
        return pltpu.make_async_copy(table_hbm.at[idx_slice], rows[b],
                                     gsems[b])

    def out_copy(c):
        b = c % NBUF
        return pltpu.make_async_copy(
            rows[b], out_hbm.at[pl.ds(base + c * CHUNK, CHUNK)], osems[b])

    for c in range(min(NBUF - 1, NCHUNK)):
        gather_copy(c).start()

    for c in range(NCHUNK):
        b = c % NBUF
        gather_copy(c).wait()

        def scale_step(i, carry, _b=b):
            r0 = i * ROWS_U
            for u in range(ROWS_U):
                for h in range(D // L):
                    v = rows[_b][r0 + u, pl.ds(h * L, L)]
                    rows[_b][r0 + u, pl.ds(h * L, L)] = v * SCALE
            return carry

        lax.fori_loop(0, CHUNK // ROWS_U, scale_step, 0)
        out_copy(c).start()

        nxt = c + NBUF - 1
        if nxt < NCHUNK:
            if nxt - NBUF >= 0:
                out_copy(nxt - NBUF).wait()
            gather_copy(nxt).start()

    for c in range(max(0, NCHUNK - NBUF), NCHUNK):
        out_copy(c).wait()


def kernel(x, table):
    xf = x.reshape(-1).astype(jnp.int32)
    mesh = plsc.VectorSubcoreMesh(core_axis_name="c", subcore_axis_name="s")
    out = pl.kernel(
        lambda t, i, o, idx_all, r0, r1, r2, g0, g1, g2, o0, o1, o2: _body(
            t, i, o, idx_all, (r0, r1, r2), (g0, g1, g2), (o0, o1, o2)),
        mesh=mesh,
        out_type=jax.ShapeDtypeStruct((B_TOTAL, D), jnp.float32),
        scratch_types=[
            pltpu.VMEM((B_PER_W,), jnp.int32),
            pltpu.VMEM((CHUNK, D), jnp.float32),
            pltpu.VMEM((CHUNK, D), jnp.float32),
            pltpu.VMEM((CHUNK, D), jnp.float32),
            pltpu.SemaphoreType.DMA,
            pltpu.SemaphoreType.DMA,
            pltpu.SemaphoreType.DMA,
            pltpu.SemaphoreType.DMA,
            pltpu.SemaphoreType.DMA,
            pltpu.SemaphoreType.DMA,
        ],
        compiler_params=pltpu.CompilerParams(use_tc_tiling_on_sc=False),
    )(table, xf)
    return out.reshape(x.shape[0], x.shape[1], D)


# speedup vs baseline: 1.4790x; 1.0014x over previous
import math

import jax
import jax.numpy as jnp
from jax import lax
from jax.experimental import pallas as pl
from jax.experimental.pallas import tpu as pltpu
from jax.experimental.pallas import tpu_sc as plsc

D = 32
SCALE = math.sqrt(32.0)

_info = plsc.get_sparse_core_info()
NC, NS, L = _info.num_cores, _info.num_subcores, _info.num_lanes
NW = NC * NS

B_TOTAL = 4096 * 200
B_PER_W = B_TOTAL // NW
NBUF = 4
CHUNK = 640
NCHUNK = B_PER_W // CHUNK
ROWS_U = 8


def _body(table_hbm, idx_hbm, out_hbm, idx_all, rows, gsems, osems):
    wid = lax.axis_index("s") * NC + lax.axis_index("c")
    base = wid * B_PER_W

    pltpu.sync_copy(idx_hbm.at[pl.ds(base, B_PER_W)], idx_all)

    def gather_copy(c):
        b = c % NBUF
        idx_slice = idx_all.at[pl.ds(c * CHUNK, CHUNK)]
        return pltpu.make_async_copy(table_hbm.at[idx_slice], rows[b],
                                     gsems[b])

    def out_copy(c):
        b = c % NBUF
        return pltpu.make_async_copy(
            rows[b], out_hbm.at[pl.ds(base + c * CHUNK, CHUNK)], osems[b])

    for c in range(min(NBUF - 1, NCHUNK)):
        gather_copy(c).start()

    for c in range(NCHUNK):
        b = c % NBUF
        gather_copy(c).wait()

        def scale_step(i, carry, _b=b):
            r0 = i * ROWS_U
            for u in range(ROWS_U):
                for h in range(D // L):
                    v = rows[_b][r0 + u, pl.ds(h * L, L)]
                    rows[_b][r0 + u, pl.ds(h * L, L)] = v * SCALE
            return carry

        lax.fori_loop(0, CHUNK // ROWS_U, scale_step, 0)
        out_copy(c).start()

        nxt = c + NBUF - 1
        if nxt < NCHUNK:
            if nxt - NBUF >= 0:
                out_copy(nxt - NBUF).wait()
            gather_copy(nxt).start()

    for c in range(max(0, NCHUNK - NBUF), NCHUNK):
        out_copy(c).wait()


def kernel(x, table):
    xf = x.reshape(-1).astype(jnp.int32)
    mesh = plsc.VectorSubcoreMesh(core_axis_name="c", subcore_axis_name="s")
    out = pl.kernel(
        lambda t, i, o, idx_all, r0, r1, r2, r3, g0, g1, g2, g3, o0, o1, o2,
        o3: _body(t, i, o, idx_all, (r0, r1, r2, r3), (g0, g1, g2, g3),
                  (o0, o1, o2, o3)),
        mesh=mesh,
        out_type=jax.ShapeDtypeStruct((B_TOTAL, D), jnp.float32),
        scratch_types=[
            pltpu.VMEM((B_PER_W,), jnp.int32),
            pltpu.VMEM((CHUNK, D), jnp.float32),
            pltpu.VMEM((CHUNK, D), jnp.float32),
            pltpu.VMEM((CHUNK, D), jnp.float32),
            pltpu.VMEM((CHUNK, D), jnp.float32),
            pltpu.SemaphoreType.DMA,
            pltpu.SemaphoreType.DMA,
            pltpu.SemaphoreType.DMA,
            pltpu.SemaphoreType.DMA,
            pltpu.SemaphoreType.DMA,
            pltpu.SemaphoreType.DMA,
            pltpu.SemaphoreType.DMA,
            pltpu.SemaphoreType.DMA,
        ],
        compiler_params=pltpu.CompilerParams(use_tc_tiling_on_sc=False),
    )(table, xf)
    return out.reshape(x.shape[0], x.shape[1], D)
